# baseline (speedup 1.0000x reference)
import functools

import jax
import jax.numpy as jnp
from jax import lax
from jax.experimental import pallas as pl
from jax.experimental.pallas import tpu as pltpu
from jax.experimental.pallas import tpu_sc as plsc

N_NODES = 50000
N_EDGES = 800000
HID = 64
HALF = 32

NC = 2
NS = 16
CHUNK = 80
E_PAD = N_EDGES
EPT = E_PAD // NS
NCHUNK = EPT // CHUNK
RING = 5
N_PAD = 51200
RPT = N_PAD // NS

BN = 2000


def _tables_body(pos_ref, we_ref, be_ref, t_ref):
    p = pos_ref[...]
    we = we_ref[...]
    af = jnp.dot(p, we[0:2, :], preferred_element_type=jnp.float32)
    bf = jnp.dot(p, we[2:4, :], preferred_element_type=jnp.float32)
    bf = bf + be_ref[...]
    t_ref[...] = jnp.concatenate([af, bf], axis=1)


def _build_tables(pos, w_edge, b_edge):
    grid = N_NODES // BN
    return pl.pallas_call(
        _tables_body,
        grid=(grid,),
        in_specs=[
            pl.BlockSpec((BN, 2), lambda i: (i, 0)),
            pl.BlockSpec((4, HID), lambda i: (0, 0)),
            pl.BlockSpec((1, HID), lambda i: (0, 0)),
        ],
        out_specs=pl.BlockSpec((BN, 2 * HID), lambda i: (i, 0)),
        out_shape=jax.ShapeDtypeStruct((N_NODES, 2 * HID), jnp.float32),
    )(pos, w_edge, b_edge.reshape(1, HID))


_MESH = plsc.VectorSubcoreMesh(core_axis_name="c", subcore_axis_name="s")


@functools.partial(
    pl.kernel,
    out_type=jax.ShapeDtypeStruct((NC, N_PAD, 4 * HALF), jnp.float32),
    mesh=_MESH,
    compiler_params=pltpu.CompilerParams(use_tc_tiling_on_sc=False,
                                         needs_layout_passes=False),
    scratch_types=[
        pltpu.VMEM((RING, 2, CHUNK), jnp.int32),
        pltpu.VMEM((RING, 2, CHUNK), jnp.int32),
        pltpu.VMEM((RING, CHUNK, HALF // 2), jnp.int32),
        pltpu.VMEM((RING, CHUNK, HALF // 2), jnp.int32),
        pltpu.VMEM((RING, CHUNK, HALF), jnp.float32),
        pltpu.VMEM_SHARED((N_PAD, HALF), jnp.float32),
    ] + [pltpu.SemaphoreType.DMA] * (4 * RING),
)
def _sc_agg(src_hbm, dst_hbm, t_hbm, out_hbm,
            idxb, adjb, msga, msgb, msg, acc,
            si0, si1, si2, si3, si4,
            sa0, sa1, sa2, sa3, sa4,
            sb0, sb1, sb2, sb3, sb4,
            ss0, ss1, ss2, ss3, ss4):
    sem_i = (si0, si1, si2, si3, si4)
    sem_a = (sa0, sa1, sa2, sa3, sa4)
    sem_b = (sb0, sb1, sb2, sb3, sb4)
    sem_s = (ss0, ss1, ss2, ss3, ss4)
    c = lax.axis_index("c")
    t = lax.axis_index("s")
    off_a = jnp.broadcast_to(c, (16,)).astype(jnp.int32)
    off_b = jnp.broadcast_to(c + 2, (16,)).astype(jnp.int32)

    zero16 = jnp.zeros((16,), jnp.float32)

    def _zfill(r, carry):
        msg[0, r, pl.ds(0, 16)] = zero16
        msg[0, r, pl.ds(16, 16)] = zero16
        return carry

    lax.fori_loop(0, CHUNK, _zfill, 0)

    def _zcopy(i, carry):
        pltpu.sync_copy(msg.at[0],
                        acc.at[pl.ds(t * RPT + i * CHUNK, CHUNK), :])
        return carry

    lax.fori_loop(0, RPT // CHUNK, _zcopy, 0)
    plsc.subcore_barrier()

    def stage_i(k, j, drain):
        if drain:
            pltpu.make_async_copy(msg.at[j], acc.at[idxb.at[j, 1]],
                                  sem_s[j]).wait()
        base = t * EPT + k * CHUNK
        pltpu.async_copy(src_hbm.at[pl.ds(base, CHUNK)], idxb.at[j, 0],
                         sem_i[j])
        pltpu.async_copy(dst_hbm.at[pl.ds(base, CHUNK)], idxb.at[j, 1],
                         sem_i[j])

    def stage_a(k, j):
        base = t * EPT + k * CHUNK
        pltpu.make_async_copy(src_hbm.at[pl.ds(base, CHUNK)], idxb.at[j, 0],
                              sem_i[j]).wait()
        pltpu.make_async_copy(dst_hbm.at[pl.ds(base, CHUNK)], idxb.at[j, 1],
                              sem_i[j]).wait()
        for i in range(CHUNK // 16):
            sl = pl.ds(i * 16, 16)
            adjb[j, 0, sl] = (idxb[j, 0, sl] << 2) + off_a
            adjb[j, 1, sl] = (idxb[j, 1, sl] << 2) + off_b
        pltpu.async_copy(t_hbm.at[adjb.at[j, 0]], msga.at[j], sem_a[j])
        pltpu.async_copy(t_hbm.at[adjb.at[j, 1]], msgb.at[j], sem_b[j])

    def stage_c(k, j):
        pltpu.make_async_copy(t_hbm.at[adjb.at[j, 0]], msga.at[j],
                              sem_a[j]).wait()
        pltpu.make_async_copy(t_hbm.at[adjb.at[j, 1]], msgb.at[j],
                              sem_b[j]).wait()
        himask = jnp.full((16,), -65536, jnp.int32)

        def _edge(r):
            wa = msga[j, r, pl.ds(0, 16)]
            wb = msgb[j, r, pl.ds(0, 16)]
            a_bf = plsc.bitcast(wa, jnp.bfloat16)
            b_bf = plsc.bitcast(wb, jnp.bfloat16)
            m = jnp.maximum(a_bf + b_bf, 0)
            w = plsc.bitcast(m, jnp.int32)
            msg[j, r, pl.ds(0, 16)] = plsc.bitcast(w << 16, jnp.float32)
            msg[j, r, pl.ds(16, 16)] = plsc.bitcast(w & himask, jnp.float32)

        plsc.parallel_loop(0, CHUNK, unroll=4)(_edge)
        pltpu.async_copy(msg.at[j], acc.at[idxb.at[j, 1]], sem_s[j],
                         add=True)

    stage_i(0, 0, False)
    stage_i(1, 1, False)
    stage_a(0, 0)

    for j in range(RING):
        k = j
        stage_i(k + 2, (j + 2) % RING, k + 2 >= RING)
        stage_a(k + 1, (j + 1) % RING)
        stage_c(k, j)

    def _outer(k5, carry):
        k0 = k5 * RING
        for j in range(RING):
            k = k0 + j
            stage_i(k + 2, (j + 2) % RING, True)
            stage_a(k + 1, (j + 1) % RING)
            stage_c(k, j)
        return carry

    lax.fori_loop(1, NCHUNK // RING - 1, _outer, 0)

    for j in range(RING):
        k = (NCHUNK - RING) + j
        if k + 2 < NCHUNK:
            stage_i(k + 2, (j + 2) % RING, True)
        if k + 1 < NCHUNK:
            stage_a(k + 1, (j + 1) % RING)
        stage_c(k, j)

    for j in range(RING):
        pltpu.make_async_copy(msg.at[j], acc.at[idxb.at[j, 1]],
                              sem_s[j]).wait()

    plsc.subcore_barrier()

    row0 = t * RPT
    pltpu.sync_copy(acc.at[pl.ds(row0, RPT), :],
                    out_hbm.at[c, pl.ds(row0, RPT), pl.ds(0, HALF)])


def _final_body(pos_ref, vel_ref, g0_ref, g1_ref,
                wpos_ref, bpos_ref, wvel_ref, bvel_ref,
                wph_ref, wvh_ref, wa0_ref, wa1_ref, bproc_ref,
                wpd_ref, bpd_ref, wvd_ref, bvd_ref,
                phat_ref, vhat_ref):
    f32 = jnp.float32
    g0 = g0_ref[0, :, 0:HALF]
    g1 = g1_ref[0, :, 0:HALF]
    ph = jnp.dot(pos_ref[...], wpos_ref[...], preferred_element_type=f32)
    ph = jnp.maximum(ph + bpos_ref[...], 0.0)
    vh = jnp.dot(vel_ref[...], wvel_ref[...], preferred_element_type=f32)
    vh = jnp.maximum(vh + bvel_ref[...], 0.0)
    h = (jnp.dot(ph, wph_ref[...], preferred_element_type=f32)
         + jnp.dot(vh, wvh_ref[...], preferred_element_type=f32)
         + jnp.dot(g0, wa0_ref[...], preferred_element_type=f32)
         + jnp.dot(g1, wa1_ref[...], preferred_element_type=f32))
    h = jnp.maximum(h + bproc_ref[...], 0.0)
    phat_ref[...] = jnp.dot(h, wpd_ref[...], preferred_element_type=f32) + bpd_ref[...]
    vhat_ref[...] = jnp.dot(h, wvd_ref[...], preferred_element_type=f32) + bvd_ref[...]


def _final_stage(pos, vel, agg, w_pos, b_pos, w_vel, b_vel,
                 w_proc, b_proc, w_pdec, b_pdec, w_vdec, b_vdec):
    grid = N_NODES // BN
    row = lambda i: (i, 0)
    rep = lambda i: (0, 0)
    pl_call = pl.pallas_call(
        _final_body,
        grid=(grid,),
        in_specs=[
            pl.BlockSpec((BN, 2), row),
            pl.BlockSpec((BN, 2), row),
            pl.BlockSpec((1, BN, 4 * HALF), lambda i: (0, i, 0)),
            pl.BlockSpec((1, BN, 4 * HALF), lambda i: (1, i, 0)),
            pl.BlockSpec((2, HID), rep),
            pl.BlockSpec((1, HID), rep),
            pl.BlockSpec((2, HID), rep),
            pl.BlockSpec((1, HID), rep),
            pl.BlockSpec((HID, HID), rep),
            pl.BlockSpec((HID, HID), rep),
            pl.BlockSpec((HALF, HID), rep),
            pl.BlockSpec((HALF, HID), rep),
            pl.BlockSpec((1, HID), rep),
            pl.BlockSpec((HID, 2), rep),
            pl.BlockSpec((1, 2), rep),
            pl.BlockSpec((HID, 2), rep),
            pl.BlockSpec((1, 2), rep),
        ],
        out_specs=[
            pl.BlockSpec((BN, 2), row),
            pl.BlockSpec((BN, 2), row),
        ],
        out_shape=[
            jax.ShapeDtypeStruct((N_NODES, 2), jnp.float32),
            jax.ShapeDtypeStruct((N_NODES, 2), jnp.float32),
        ],
    )
    perm = jnp.array([i for i in range(0, HALF, 2)]
                     + [i for i in range(1, HALF, 2)], dtype=jnp.int32)
    wa0 = w_proc[2 * HID:2 * HID + HALF][perm]
    wa1 = w_proc[2 * HID + HALF:][perm]
    return pl_call(pos, vel, agg, agg,
                   w_pos, b_pos.reshape(1, HID), w_vel, b_vel.reshape(1, HID),
                   w_proc[0:HID], w_proc[HID:2 * HID], wa0, wa1,
                   b_proc.reshape(1, HID),
                   w_pdec, b_pdec.reshape(1, 2), w_vdec, b_vdec.reshape(1, 2))


def kernel(pos, vel, edge_index, W_pos, b_pos, W_vel, b_vel, W_edge, b_edge,
           W_proc, b_proc, W_pdec, b_pdec, W_vdec, b_vdec):
    ei = edge_index.astype(jnp.int32)
    src = ei[0]
    dst = ei[1]

    table = _build_tables(pos, W_edge, b_edge)
    table_bf = table.astype(jnp.bfloat16).reshape(N_NODES, 2 * HID // 2, 2)
    table_w = lax.bitcast_convert_type(table_bf, jnp.int32)
    table4 = table_w.reshape(4 * N_NODES, HALF // 2)

    agg = _sc_agg(src, dst, table4)

    pos_hat, vel_hat = _final_stage(
        pos, vel, agg, W_pos, b_pos, W_vel, b_vel,
        W_proc, b_proc, W_pdec, b_pdec, W_vdec, b_vdec)
    return (pos_hat, vel_hat)


# speedup vs baseline: 1.3774x; 1.3774x over previous
import functools

import jax
import jax.numpy as jnp
from jax import lax
from jax.experimental import pallas as pl
from jax.experimental.pallas import tpu as pltpu
from jax.experimental.pallas import tpu_sc as plsc

N_NODES = 50000
N_EDGES = 800000
HID = 64
HALF = 32

NC = 2
NS = 16
CHUNK = 80
E_PAD = N_EDGES
EPT = E_PAD // NS
NCHUNK = EPT // CHUNK
RING = 5
N_PAD = 51200
RPT = N_PAD // NS

BN = 2000


def _tables_body(pos_ref, we_ref, be_ref, t_ref):
    p = pos_ref[...]
    we = we_ref[...]
    af = jnp.dot(p, we[0:2, :], preferred_element_type=jnp.float32)
    bf = jnp.dot(p, we[2:4, :], preferred_element_type=jnp.float32)
    bf = bf + be_ref[...]
    t_ref[...] = jnp.concatenate([af, bf], axis=1)


def _build_tables(pos, w_edge, b_edge):
    grid = N_NODES // BN
    return pl.pallas_call(
        _tables_body,
        grid=(grid,),
        in_specs=[
            pl.BlockSpec((BN, 2), lambda i: (i, 0)),
            pl.BlockSpec((4, HID), lambda i: (0, 0)),
            pl.BlockSpec((1, HID), lambda i: (0, 0)),
        ],
        out_specs=pl.BlockSpec((BN, 2 * HID), lambda i: (i, 0)),
        out_shape=jax.ShapeDtypeStruct((N_NODES, 2 * HID), jnp.float32),
    )(pos, w_edge, b_edge.reshape(1, HID))


_MESH = plsc.VectorSubcoreMesh(core_axis_name="c", subcore_axis_name="s")


@functools.partial(
    pl.kernel,
    out_type=jax.ShapeDtypeStruct((NC, N_PAD, 4 * HALF), jnp.float32),
    mesh=_MESH,
    compiler_params=pltpu.CompilerParams(use_tc_tiling_on_sc=False),
    scratch_types=[
        pltpu.VMEM((RING, 2, CHUNK), jnp.int32),
        pltpu.VMEM((RING, 2, CHUNK), jnp.int32),
        pltpu.VMEM((RING, CHUNK, HALF), jnp.float32),
        pltpu.VMEM_SHARED((N_PAD, HALF), jnp.float32),
    ] + [pltpu.SemaphoreType.DMA] * (4 * RING),
)
def _sc_agg(src_hbm, dst_hbm, t_hbm, out_hbm,
            idxb, adjb, msg, acc,
            si0, si1, si2, si3, si4,
            sa0, sa1, sa2, sa3, sa4,
            sb0, sb1, sb2, sb3, sb4,
            ss0, ss1, ss2, ss3, ss4):
    sem_i = (si0, si1, si2, si3, si4)
    sem_a = (sa0, sa1, sa2, sa3, sa4)
    sem_b = (sb0, sb1, sb2, sb3, sb4)
    sem_s = (ss0, ss1, ss2, ss3, ss4)
    c = lax.axis_index("c")
    t = lax.axis_index("s")
    off_a = jnp.broadcast_to(c, (16,)).astype(jnp.int32)
    off_b = jnp.broadcast_to(c + 2, (16,)).astype(jnp.int32)

    zero16 = jnp.zeros((16,), jnp.float32)

    def _zfill(r, carry):
        msg[0, r, pl.ds(0, 16)] = zero16
        msg[0, r, pl.ds(16, 16)] = zero16
        return carry

    lax.fori_loop(0, CHUNK, _zfill, 0)

    def _zcopy(i, carry):
        pltpu.sync_copy(msg.at[0],
                        acc.at[pl.ds(t * RPT + i * CHUNK, CHUNK), :])
        return carry

    lax.fori_loop(0, RPT // CHUNK, _zcopy, 0)
    plsc.subcore_barrier()

    def stage_i(k, j, drain):
        if drain:
            pltpu.make_async_copy(msg.at[j], acc.at[idxb.at[j, 1]],
                                  sem_s[j]).wait()
        base = t * EPT + k * CHUNK
        pltpu.async_copy(src_hbm.at[pl.ds(base, CHUNK)], idxb.at[j, 0],
                         sem_i[j])
        pltpu.async_copy(dst_hbm.at[pl.ds(base, CHUNK)], idxb.at[j, 1],
                         sem_i[j])

    def stage_a(k, j):
        base = t * EPT + k * CHUNK
        pltpu.make_async_copy(src_hbm.at[pl.ds(base, CHUNK)], idxb.at[j, 0],
                              sem_i[j]).wait()
        pltpu.make_async_copy(dst_hbm.at[pl.ds(base, CHUNK)], idxb.at[j, 1],
                              sem_i[j]).wait()
        for i in range(CHUNK // 16):
            sl = pl.ds(i * 16, 16)
            adjb[j, 0, sl] = (idxb[j, 0, sl] << 2) + off_a
            adjb[j, 1, sl] = (idxb[j, 1, sl] << 2) + off_b
        pltpu.async_copy(t_hbm.at[adjb.at[j, 0]], msg.at[j], sem_a[j])

    def stage_b(k, j):
        pltpu.make_async_copy(t_hbm.at[adjb.at[j, 0]], msg.at[j],
                              sem_a[j]).wait()
        pltpu.async_copy(t_hbm.at[adjb.at[j, 1]], msg.at[j], sem_b[j],
                         add=True)

    def stage_c(k, j):
        pltpu.make_async_copy(t_hbm.at[adjb.at[j, 1]], msg.at[j],
                              sem_b[j]).wait()

        def _relu(r):
            msg[j, r, pl.ds(0, 16)] = jnp.maximum(msg[j, r, pl.ds(0, 16)], 0.0)
            msg[j, r, pl.ds(16, 16)] = jnp.maximum(msg[j, r, pl.ds(16, 16)], 0.0)

        plsc.parallel_loop(0, CHUNK, unroll=4)(_relu)
        pltpu.async_copy(msg.at[j], acc.at[idxb.at[j, 1]], sem_s[j],
                         add=True)

    stage_i(0, 0, False)
    stage_i(1, 1, False)
    stage_i(2, 2, False)
    stage_a(0, 0)
    stage_a(1, 1)
    stage_b(0, 0)

    for j in range(RING):
        k = j
        stage_i(k + 3, (j + 3) % RING, k + 3 >= RING)
        stage_a(k + 2, (j + 2) % RING)
        stage_b(k + 1, (j + 1) % RING)
        stage_c(k, j)

    def _outer(k5, carry):
        k0 = k5 * RING
        for j in range(RING):
            k = k0 + j
            stage_i(k + 3, (j + 3) % RING, True)
            stage_a(k + 2, (j + 2) % RING)
            stage_b(k + 1, (j + 1) % RING)
            stage_c(k, j)
        return carry

    lax.fori_loop(1, NCHUNK // RING - 1, _outer, 0)

    for j in range(RING):
        k = (NCHUNK - RING) + j
        if k + 3 < NCHUNK:
            stage_i(k + 3, (j + 3) % RING, True)
        if k + 2 < NCHUNK:
            stage_a(k + 2, (j + 2) % RING)
        if k + 1 < NCHUNK:
            stage_b(k + 1, (j + 1) % RING)
        stage_c(k, j)

    for j in range(RING):
        pltpu.make_async_copy(msg.at[j], acc.at[idxb.at[j, 1]],
                              sem_s[j]).wait()

    plsc.subcore_barrier()

    row0 = t * RPT
    pltpu.sync_copy(acc.at[pl.ds(row0, RPT), :],
                    out_hbm.at[c, pl.ds(row0, RPT), pl.ds(0, HALF)])


def _final_body(pos_ref, vel_ref, g0_ref, g1_ref,
                wpos_ref, bpos_ref, wvel_ref, bvel_ref,
                wph_ref, wvh_ref, wa0_ref, wa1_ref, bproc_ref,
                wpd_ref, bpd_ref, wvd_ref, bvd_ref,
                phat_ref, vhat_ref):
    f32 = jnp.float32
    g0 = g0_ref[0, :, 0:HALF]
    g1 = g1_ref[0, :, 0:HALF]
    ph = jnp.dot(pos_ref[...], wpos_ref[...], preferred_element_type=f32)
    ph = jnp.maximum(ph + bpos_ref[...], 0.0)
    vh = jnp.dot(vel_ref[...], wvel_ref[...], preferred_element_type=f32)
    vh = jnp.maximum(vh + bvel_ref[...], 0.0)
    h = (jnp.dot(ph, wph_ref[...], preferred_element_type=f32)
         + jnp.dot(vh, wvh_ref[...], preferred_element_type=f32)
         + jnp.dot(g0, wa0_ref[...], preferred_element_type=f32)
         + jnp.dot(g1, wa1_ref[...], preferred_element_type=f32))
    h = jnp.maximum(h + bproc_ref[...], 0.0)
    phat_ref[...] = jnp.dot(h, wpd_ref[...], preferred_element_type=f32) + bpd_ref[...]
    vhat_ref[...] = jnp.dot(h, wvd_ref[...], preferred_element_type=f32) + bvd_ref[...]


def _final_stage(pos, vel, agg, w_pos, b_pos, w_vel, b_vel,
                 w_proc, b_proc, w_pdec, b_pdec, w_vdec, b_vdec):
    grid = N_NODES // BN
    row = lambda i: (i, 0)
    rep = lambda i: (0, 0)
    pl_call = pl.pallas_call(
        _final_body,
        grid=(grid,),
        in_specs=[
            pl.BlockSpec((BN, 2), row),
            pl.BlockSpec((BN, 2), row),
            pl.BlockSpec((1, BN, 4 * HALF), lambda i: (0, i, 0)),
            pl.BlockSpec((1, BN, 4 * HALF), lambda i: (1, i, 0)),
            pl.BlockSpec((2, HID), rep),
            pl.BlockSpec((1, HID), rep),
            pl.BlockSpec((2, HID), rep),
            pl.BlockSpec((1, HID), rep),
            pl.BlockSpec((HID, HID), rep),
            pl.BlockSpec((HID, HID), rep),
            pl.BlockSpec((HALF, HID), rep),
            pl.BlockSpec((HALF, HID), rep),
            pl.BlockSpec((1, HID), rep),
            pl.BlockSpec((HID, 2), rep),
            pl.BlockSpec((1, 2), rep),
            pl.BlockSpec((HID, 2), rep),
            pl.BlockSpec((1, 2), rep),
        ],
        out_specs=[
            pl.BlockSpec((BN, 2), row),
            pl.BlockSpec((BN, 2), row),
        ],
        out_shape=[
            jax.ShapeDtypeStruct((N_NODES, 2), jnp.float32),
            jax.ShapeDtypeStruct((N_NODES, 2), jnp.float32),
        ],
    )
    wa0 = w_proc[2 * HID:2 * HID + HALF]
    wa1 = w_proc[2 * HID + HALF:]
    return pl_call(pos, vel, agg, agg,
                   w_pos, b_pos.reshape(1, HID), w_vel, b_vel.reshape(1, HID),
                   w_proc[0:HID], w_proc[HID:2 * HID], wa0, wa1,
                   b_proc.reshape(1, HID),
                   w_pdec, b_pdec.reshape(1, 2), w_vdec, b_vdec.reshape(1, 2))


def kernel(pos, vel, edge_index, W_pos, b_pos, W_vel, b_vel, W_edge, b_edge,
           W_proc, b_proc, W_pdec, b_pdec, W_vdec, b_vdec):
    ei = edge_index.astype(jnp.int32)
    src = ei[0]
    dst = ei[1]

    table = _build_tables(pos, W_edge, b_edge)
    table4 = table.reshape(4 * N_NODES, HALF)

    agg = _sc_agg(src, dst, table4)

    pos_hat, vel_hat = _final_stage(
        pos, vel, agg, W_pos, b_pos, W_vel, b_vel,
        W_proc, b_proc, W_pdec, b_pdec, W_vdec, b_vdec)
    return (pos_hat, vel_hat)


# speedup vs baseline: 1.4426x; 1.0473x over previous
import functools

import jax
import jax.numpy as jnp
from jax import lax
from jax.experimental import pallas as pl
from jax.experimental.pallas import tpu as pltpu
from jax.experimental.pallas import tpu_sc as plsc

N_NODES = 50000
N_EDGES = 800000
HID = 64
HALF = 32

NC = 2
NS = 16
CHUNK = 80
E_PAD = N_EDGES
EPT = E_PAD // NS
NCHUNK = EPT // CHUNK
RING = 5
N_PAD = 51200
RPT = N_PAD // NS

BN = 2000


def _tables_body(pos_ref, we_ref, be_ref, t_ref):
    p = pos_ref[...]
    we = we_ref[...]
    af = jnp.dot(p, we[0:2, :], preferred_element_type=jnp.float32)
    bf = jnp.dot(p, we[2:4, :], preferred_element_type=jnp.float32)
    bf = bf + be_ref[...]
    t_ref[...] = jnp.concatenate([af, bf], axis=1)


def _build_tables(pos, w_edge, b_edge):
    grid = N_NODES // BN
    return pl.pallas_call(
        _tables_body,
        grid=(grid,),
        in_specs=[
            pl.BlockSpec((BN, 2), lambda i: (i, 0)),
            pl.BlockSpec((4, HID), lambda i: (0, 0)),
            pl.BlockSpec((1, HID), lambda i: (0, 0)),
        ],
        out_specs=pl.BlockSpec((BN, 2 * HID), lambda i: (i, 0)),
        out_shape=jax.ShapeDtypeStruct((N_NODES, 2 * HID), jnp.float32),
    )(pos, w_edge, b_edge.reshape(1, HID))


_MESH = plsc.VectorSubcoreMesh(core_axis_name="c", subcore_axis_name="s")


@functools.partial(
    pl.kernel,
    out_type=jax.ShapeDtypeStruct((NC, N_PAD, 4 * HALF), jnp.float32),
    mesh=_MESH,
    compiler_params=pltpu.CompilerParams(use_tc_tiling_on_sc=False),
    scratch_types=[
        pltpu.VMEM((RING, 2, CHUNK), jnp.int32),
        pltpu.VMEM((RING, 2, CHUNK), jnp.int32),
        pltpu.VMEM((RING, CHUNK, HALF), jnp.float32),
        pltpu.VMEM_SHARED((N_PAD, HALF), jnp.float32),
    ] + [pltpu.SemaphoreType.DMA] * (4 * RING),
)
def _sc_agg(ei_hbm, t_hbm, out_hbm,
            idxb, adjb, msg, acc,
            si0, si1, si2, si3, si4,
            sa0, sa1, sa2, sa3, sa4,
            sb0, sb1, sb2, sb3, sb4,
            ss0, ss1, ss2, ss3, ss4):
    sem_i = (si0, si1, si2, si3, si4)
    sem_a = (sa0, sa1, sa2, sa3, sa4)
    sem_b = (sb0, sb1, sb2, sb3, sb4)
    sem_s = (ss0, ss1, ss2, ss3, ss4)
    c = lax.axis_index("c")
    t = lax.axis_index("s")
    off_a = jnp.broadcast_to(c, (16,)).astype(jnp.int32)
    off_b = jnp.broadcast_to(c + 2, (16,)).astype(jnp.int32)

    zero16 = jnp.zeros((16,), jnp.float32)

    def _zfill(r, carry):
        msg[0, r, pl.ds(0, 16)] = zero16
        msg[0, r, pl.ds(16, 16)] = zero16
        return carry

    lax.fori_loop(0, CHUNK, _zfill, 0)

    def _zcopy(i, carry):
        pltpu.sync_copy(msg.at[0],
                        acc.at[pl.ds(t * RPT + i * CHUNK, CHUNK), :])
        return carry

    lax.fori_loop(0, RPT // CHUNK, _zcopy, 0)
    plsc.subcore_barrier()

    def stage_i(k, j, drain):
        if drain:
            pltpu.make_async_copy(msg.at[j], acc.at[idxb.at[j, 1]],
                                  sem_s[j]).wait()
        base = t * EPT + k * CHUNK
        pltpu.async_copy(ei_hbm.at[:, pl.ds(base, CHUNK)], idxb.at[j],
                         sem_i[j])

    def stage_a(k, j):
        base = t * EPT + k * CHUNK
        pltpu.make_async_copy(ei_hbm.at[:, pl.ds(base, CHUNK)], idxb.at[j],
                              sem_i[j]).wait()
        for i in range(CHUNK // 16):
            sl = pl.ds(i * 16, 16)
            adjb[j, 0, sl] = (idxb[j, 0, sl] << 2) + off_a
            adjb[j, 1, sl] = (idxb[j, 1, sl] << 2) + off_b
        pltpu.async_copy(t_hbm.at[adjb.at[j, 0]], msg.at[j], sem_a[j])

    def stage_b(k, j):
        pltpu.make_async_copy(t_hbm.at[adjb.at[j, 0]], msg.at[j],
                              sem_a[j]).wait()
        pltpu.async_copy(t_hbm.at[adjb.at[j, 1]], msg.at[j], sem_b[j],
                         add=True)

    def stage_c(k, j):
        pltpu.make_async_copy(t_hbm.at[adjb.at[j, 1]], msg.at[j],
                              sem_b[j]).wait()

        def _relu(r):
            msg[j, r, pl.ds(0, 16)] = jnp.maximum(msg[j, r, pl.ds(0, 16)], 0.0)
            msg[j, r, pl.ds(16, 16)] = jnp.maximum(msg[j, r, pl.ds(16, 16)], 0.0)

        plsc.parallel_loop(0, CHUNK, unroll=4)(_relu)
        pltpu.async_copy(msg.at[j], acc.at[idxb.at[j, 1]], sem_s[j],
                         add=True)

    stage_i(0, 0, False)
    stage_i(1, 1, False)
    stage_i(2, 2, False)
    stage_a(0, 0)
    stage_a(1, 1)
    stage_b(0, 0)

    for j in range(RING):
        k = j
        stage_i(k + 3, (j + 3) % RING, k + 3 >= RING)
        stage_a(k + 2, (j + 2) % RING)
        stage_b(k + 1, (j + 1) % RING)
        stage_c(k, j)

    def _outer(k5, carry):
        k0 = k5 * RING
        for j in range(RING):
            k = k0 + j
            stage_i(k + 3, (j + 3) % RING, True)
            stage_a(k + 2, (j + 2) % RING)
            stage_b(k + 1, (j + 1) % RING)
            stage_c(k, j)
        return carry

    lax.fori_loop(1, NCHUNK // RING - 1, _outer, 0)

    for j in range(RING):
        k = (NCHUNK - RING) + j
        if k + 3 < NCHUNK:
            stage_i(k + 3, (j + 3) % RING, True)
        if k + 2 < NCHUNK:
            stage_a(k + 2, (j + 2) % RING)
        if k + 1 < NCHUNK:
            stage_b(k + 1, (j + 1) % RING)
        stage_c(k, j)

    for j in range(RING):
        pltpu.make_async_copy(msg.at[j], acc.at[idxb.at[j, 1]],
                              sem_s[j]).wait()

    plsc.subcore_barrier()

    row0 = t * RPT
    pltpu.sync_copy(acc.at[pl.ds(row0, RPT), :],
                    out_hbm.at[c, pl.ds(row0, RPT), pl.ds(0, HALF)])


def _final_body(pos_ref, vel_ref, g0_ref, g1_ref,
                wpos_ref, bpos_ref, wvel_ref, bvel_ref,
                wph_ref, wvh_ref, wa0_ref, wa1_ref, bproc_ref,
                wpd_ref, bpd_ref, wvd_ref, bvd_ref,
                phat_ref, vhat_ref):
    f32 = jnp.float32
    g0 = g0_ref[0, :, 0:HALF]
    g1 = g1_ref[0, :, 0:HALF]
    ph = jnp.dot(pos_ref[...], wpos_ref[...], preferred_element_type=f32)
    ph = jnp.maximum(ph + bpos_ref[...], 0.0)
    vh = jnp.dot(vel_ref[...], wvel_ref[...], preferred_element_type=f32)
    vh = jnp.maximum(vh + bvel_ref[...], 0.0)
    h = (jnp.dot(ph, wph_ref[...], preferred_element_type=f32)
         + jnp.dot(vh, wvh_ref[...], preferred_element_type=f32)
         + jnp.dot(g0, wa0_ref[...], preferred_element_type=f32)
         + jnp.dot(g1, wa1_ref[...], preferred_element_type=f32))
    h = jnp.maximum(h + bproc_ref[...], 0.0)
    phat_ref[...] = jnp.dot(h, wpd_ref[...], preferred_element_type=f32) + bpd_ref[...]
    vhat_ref[...] = jnp.dot(h, wvd_ref[...], preferred_element_type=f32) + bvd_ref[...]


def _final_stage(pos, vel, agg, w_pos, b_pos, w_vel, b_vel,
                 w_proc, b_proc, w_pdec, b_pdec, w_vdec, b_vdec):
    grid = N_NODES // BN
    row = lambda i: (i, 0)
    rep = lambda i: (0, 0)
    pl_call = pl.pallas_call(
        _final_body,
        grid=(grid,),
        in_specs=[
            pl.BlockSpec((BN, 2), row),
            pl.BlockSpec((BN, 2), row),
            pl.BlockSpec((1, BN, 4 * HALF), lambda i: (0, i, 0)),
            pl.BlockSpec((1, BN, 4 * HALF), lambda i: (1, i, 0)),
            pl.BlockSpec((2, HID), rep),
            pl.BlockSpec((1, HID), rep),
            pl.BlockSpec((2, HID), rep),
            pl.BlockSpec((1, HID), rep),
            pl.BlockSpec((HID, HID), rep),
            pl.BlockSpec((HID, HID), rep),
            pl.BlockSpec((HALF, HID), rep),
            pl.BlockSpec((HALF, HID), rep),
            pl.BlockSpec((1, HID), rep),
            pl.BlockSpec((HID, 2), rep),
            pl.BlockSpec((1, 2), rep),
            pl.BlockSpec((HID, 2), rep),
            pl.BlockSpec((1, 2), rep),
        ],
        out_specs=[
            pl.BlockSpec((BN, 2), row),
            pl.BlockSpec((BN, 2), row),
        ],
        out_shape=[
            jax.ShapeDtypeStruct((N_NODES, 2), jnp.float32),
            jax.ShapeDtypeStruct((N_NODES, 2), jnp.float32),
        ],
    )
    wa0 = w_proc[2 * HID:2 * HID + HALF]
    wa1 = w_proc[2 * HID + HALF:]
    return pl_call(pos, vel, agg, agg,
                   w_pos, b_pos.reshape(1, HID), w_vel, b_vel.reshape(1, HID),
                   w_proc[0:HID], w_proc[HID:2 * HID], wa0, wa1,
                   b_proc.reshape(1, HID),
                   w_pdec, b_pdec.reshape(1, 2), w_vdec, b_vdec.reshape(1, 2))


def kernel(pos, vel, edge_index, W_pos, b_pos, W_vel, b_vel, W_edge, b_edge,
           W_proc, b_proc, W_pdec, b_pdec, W_vdec, b_vdec):
    ei = edge_index.astype(jnp.int32)

    table = _build_tables(pos, W_edge, b_edge)
    table4 = table.reshape(4 * N_NODES, HALF)

    agg = _sc_agg(ei, table4)

    pos_hat, vel_hat = _final_stage(
        pos, vel, agg, W_pos, b_pos, W_vel, b_vel,
        W_proc, b_proc, W_pdec, b_pdec, W_vdec, b_vdec)
    return (pos_hat, vel_hat)
